# baseline (speedup 1.0000x reference)
import functools

import jax
import jax.numpy as jnp
from jax.experimental import pallas as pl
from jax.experimental.pallas import tpu as pltpu
from jax.experimental.pallas import tpu_sc as plsc

D = 768
H = 3072
E = 8
S = 2048
EPAD = 128
NEG = -1e30

BLK = 512
G = 16
P = G * BLK

NC, NS = 2, 16
NW = NC * NS
CHUNK = S // NW

@functools.cache
def _sc_mesh():
    return plsc.VectorSubcoreMesh(core_axis_name="c", subcore_axis_name="s",
                                  num_cores=NC, num_subcores=NS)


def _route_kernel(x_ref, gw_ref, gb_ref, posw_ref, probw_ref, meta_ref):
    x = x_ref[...]
    scores = jnp.dot(x, gw_ref[...], preferred_element_type=jnp.float32) + gb_ref[...]
    lane = jax.lax.broadcasted_iota(jnp.int32, (S, EPAD), 1)
    m1 = jnp.max(scores, axis=1, keepdims=True)
    i1 = jnp.min(jnp.where(scores == m1, lane, EPAD), axis=1, keepdims=True)
    masked = jnp.where(lane == i1, NEG, scores)
    m2 = jnp.max(masked, axis=1, keepdims=True)
    i2 = jnp.min(jnp.where(masked == m2, lane, EPAD), axis=1, keepdims=True)
    e2 = jnp.exp(m2 - m1)
    p1 = 1.0 / (1.0 + e2)
    p2 = e2 / (1.0 + e2)

    oh = (jnp.where(lane == i1, 1.0, 0.0)
          + jnp.where(lane == i2 + E, 1.0, 0.0))
    r_i = jax.lax.broadcasted_iota(jnp.int32, (S, S), 0)
    c_i = jax.lax.broadcasted_iota(jnp.int32, (S, S), 1)
    tril = jnp.where(r_i >= c_i, 1.0, 0.0)
    csum = jnp.dot(tril, oh, preferred_element_type=jnp.float32)
    lane1 = jax.lax.broadcasted_iota(jnp.int32, (1, EPAD), 1)
    tot = csum[S - 1:S, :]
    t0 = jnp.where(lane1 < E, tot, 0.0)
    t1s = jnp.pad(tot[:, E:], ((0, 0), (0, E)))
    cnt = t0 + jnp.where(lane1 < E, t1s, 0.0)

    cnt_i = cnt.astype(jnp.int32)
    pc_i = ((cnt_i + (BLK - 1)) >> 9) << 9
    pc = pc_i.astype(jnp.float32)
    ru = jax.lax.broadcasted_iota(jnp.int32, (EPAD, EPAD), 0)
    cu = jax.lax.broadcasted_iota(jnp.int32, (EPAD, EPAD), 1)
    triu = jnp.where(ru < cu, 1.0, 0.0)
    off = jnp.dot(pc, triu, preferred_element_type=jnp.float32)
    cum = off + pc

    rank0 = jnp.sum(jnp.where(lane == i1, csum, 0.0), axis=1, keepdims=True) - 1.0
    rank1 = jnp.sum(jnp.where(lane == i2 + E, csum, 0.0), axis=1, keepdims=True) - 1.0
    off_b = jnp.broadcast_to(off, (S, EPAD))
    t0_b = jnp.broadcast_to(t0, (S, EPAD))
    off0 = jnp.sum(jnp.where(lane == i1, off_b, 0.0), axis=1, keepdims=True)
    off1 = jnp.sum(jnp.where(lane == i2, off_b, 0.0), axis=1, keepdims=True)
    t0e1 = jnp.sum(jnp.where(lane == i2, t0_b, 0.0), axis=1, keepdims=True)
    pos0 = off0 + rank0
    pos1 = off1 + t0e1 + rank1
    posw_ref[...] = jnp.where(
        lane == 0, pos0, jnp.where(lane == 1, pos1, 0.0)).astype(jnp.int32)
    probw_ref[...] = jnp.where(
        lane < 16, p1, jnp.where(lane < 32, p2, 0.0))

    total_i = jnp.sum(pc_i, axis=1, keepdims=True)
    gclamp = jnp.minimum(lane1 * BLK, total_i - 1)
    cum_i = cum.astype(jnp.int32)
    be = jnp.zeros((1, EPAD), jnp.int32)
    for e in range(E):
        be = be + jnp.where(cum_i[:, e:e + 1] <= gclamp, 1, 0)
    be = jnp.minimum(be, E - 1)
    nact = total_i >> 9
    brow = jnp.minimum(lane1, nact - 1)

    be_f = be.astype(jnp.float32)
    be_sh = jnp.pad(be_f[:, :EPAD - 1], ((0, 0), (1, 0)), constant_values=-1.0)
    switch_g = jnp.where(be_f != be_sh, 1.0, 0.0)
    tril_incl = jnp.where(ru <= cu, 1.0, 0.0)
    s_g = jnp.dot(switch_g, tril_incl,
                  preferred_element_type=jnp.float32) - 1.0
    nxt = jnp.full((1, EPAD), -1.0)
    for e in range(E - 1, -1, -1):
        pres_e = pc_i[:, e:e + 1] > 0
        nxt = jnp.where(jnp.logical_and(pres_e, be < e), float(e), nxt)
    m2 = jnp.where(ru == (cu >> 1), 1.0, 0.0)
    dotm2 = lambda v: jnp.dot(v, m2, preferred_element_type=jnp.float32)
    s_ti = dotm2(s_g).astype(jnp.int32)
    dwait = dotm2(switch_g).astype(jnp.int32)
    nxt_t = dotm2(nxt).astype(jnp.int32)
    p_t = lane1 & 1
    uslot = (2 * s_ti + p_t) & 3
    islot = (2 * s_ti + 2 + p_t) & 3
    ieid = jnp.where((dwait == 1) & (s_ti >= 1), nxt_t, -1)
    nxt_g = nxt.astype(jnp.int32)

    rows = [be, brow, nact, uslot, dwait, ieid, islot, nxt_g]
    row = jax.lax.broadcasted_iota(jnp.int32, (8, EPAD), 0)
    out = jnp.broadcast_to(rows[7], (8, EPAD))
    for r in range(6, -1, -1):
        out = jnp.where(row == r, jnp.broadcast_to(rows[r], (8, EPAD)), out)
    meta_ref[...] = out


def _scatter_body(x_hbm, pos_hbm, xs_hbm, xv, iv, sem):
    wid = jax.lax.axis_index("s") * NC + jax.lax.axis_index("c")
    base = wid * CHUNK
    pltpu.sync_copy(x_hbm.at[pl.ds(base, CHUNK)], xv)
    for k in range(2):
        pltpu.sync_copy(pos_hbm.at[k, pl.ds(base, CHUNK)], iv)
        pltpu.async_copy(xv, xs_hbm.at[iv], sem).wait()


H2 = H // 2


def _ffn_kernel(be_s, brow_s, nact_s, uslot_s, dwait_s, ieid_s, islot_s,
                nxtg_s, xs_ref, b1_ref, b2_ref, w1_hbm, w2_hbm, y_ref,
                w1b, w2b, sem1, sem2):
    g = pl.program_id(0)
    p = pl.program_id(1)
    t = 2 * g + p

    def unit_copies(eid, ph, slot):
        c1 = pltpu.make_async_copy(
            w1_hbm.at[eid, :, pl.ds(ph * H2, H2)], w1b.at[slot], sem1.at[slot])
        c2 = pltpu.make_async_copy(
            w2_hbm.at[eid, pl.ds(ph * H2, H2), :], w2b.at[slot], sem2.at[slot])
        return c1, c2

    @pl.when(t == 0)
    def _():
        for ph in (0, 1):
            c1, c2 = unit_copies(be_s[0], ph, ph)
            c1.start()
            c2.start()

        @pl.when(nxtg_s[0] >= 0)
        def _():
            for ph in (0, 1):
                c1, c2 = unit_copies(nxtg_s[0], ph, 2 + ph)
                c1.start()
                c2.start()

    @pl.when(ieid_s[t] >= 0)
    def _():
        c1, c2 = unit_copies(ieid_s[t], p, islot_s[t])
        c1.start()
        c2.start()

    @pl.when(dwait_s[t] == 1)
    def _():
        c1, c2 = unit_copies(0, 0, uslot_s[t])
        c1.wait()
        c2.wait()

    @pl.when(g < nact_s[0])
    def _():
        slot = uslot_s[t]
        hid = jnp.dot(xs_ref[...], w1b[slot],
                      preferred_element_type=jnp.float32) + b1_ref[0, 0]
        hid = jnp.maximum(hid, 0.0)
        part = jnp.dot(hid, w2b[slot], preferred_element_type=jnp.float32)

        @pl.when(p == 0)
        def _():
            y_ref[...] = part + b2_ref[0, 0]

        @pl.when(p == 1)
        def _():
            y_ref[...] += part


def _combine_body(ys_hbm, pos_hbm, probw_hbm, out_hbm, i0v, i1v, y0v, y1v,
                  qwv, sem):
    wid = jax.lax.axis_index("s") * NC + jax.lax.axis_index("c")
    base = wid * CHUNK
    pltpu.sync_copy(pos_hbm.at[0, pl.ds(base, CHUNK)], i0v)
    pltpu.sync_copy(pos_hbm.at[1, pl.ds(base, CHUNK)], i1v)
    pltpu.sync_copy(probw_hbm.at[pl.ds(base * EPAD, CHUNK * EPAD)], qwv)
    cp0 = pltpu.async_copy(ys_hbm.at[i0v], y0v, sem)
    cp1 = pltpu.async_copy(ys_hbm.at[i1v], y1v, sem)
    cp0.wait()
    cp1.wait()

    def row_add(r, carry):
        p0 = qwv[pl.ds(r * EPAD, 16)]
        p1 = qwv[pl.ds(r * EPAD + 16, 16)]
        for j in range(D // 16):
            sl = pl.ds(j * 16, 16)
            y0v[r, sl] = y0v[r, sl] * p0 + y1v[r, sl] * p1
        return carry

    jax.lax.fori_loop(0, CHUNK, row_add, 0)
    pltpu.sync_copy(y0v, out_hbm.at[pl.ds(base, CHUNK)])


def kernel(x, gate_W, gate_b, W1, b1, W2, b2):
    x2d = x.reshape(S, D)
    gate_Wp = jnp.pad(gate_W, ((0, 0), (0, EPAD - E)))
    gate_bp = jnp.pad(gate_b.reshape(1, E), ((0, 0), (0, EPAD - E)),
                      constant_values=NEG)

    posw, probw, meta = pl.pallas_call(
        _route_kernel,
        out_shape=(
            jax.ShapeDtypeStruct((S, EPAD), jnp.int32),
            jax.ShapeDtypeStruct((S, EPAD), jnp.float32),
            jax.ShapeDtypeStruct((8, EPAD), jnp.int32),
        ),
    )(x2d, gate_Wp, gate_bp)

    pos = jnp.stack([posw[:, 0], posw[:, 1]])
    be = meta[0, :G]
    brow = meta[1, :G]
    nact = meta[2, :1]
    uslot = meta[3, :2 * G]
    dwait = meta[4, :2 * G]
    ieid = meta[5, :2 * G]
    islot = meta[6, :2 * G]
    nxtg = meta[7, :1]

    xs = pl.kernel(
        _scatter_body,
        out_type=jax.ShapeDtypeStruct((P, D), jnp.float32),
        mesh=_sc_mesh(),
        scratch_types=[
            pltpu.VMEM((CHUNK, D), jnp.float32),
            pltpu.VMEM((CHUNK,), jnp.int32),
            pltpu.SemaphoreType.DMA,
        ],
    )(x2d, pos)

    grid_spec = pltpu.PrefetchScalarGridSpec(
        num_scalar_prefetch=8,
        grid=(G, 2),
        in_specs=[
            pl.BlockSpec((BLK, D), lambda g, p, be, br, *_: (br[g], 0)),
            pl.BlockSpec((1, 1, H2), lambda g, p, be, br, *_: (be[g], 0, p)),
            pl.BlockSpec((1, 1, D), lambda g, p, be, br, *_: (be[g], 0, 0)),
            pl.BlockSpec(memory_space=pl.ANY),
            pl.BlockSpec(memory_space=pl.ANY),
        ],
        out_specs=pl.BlockSpec((BLK, D), lambda g, p, be, br, *_: (br[g], 0)),
        scratch_shapes=[
            pltpu.VMEM((4, D, H2), jnp.float32),
            pltpu.VMEM((4, H2, D), jnp.float32),
            pltpu.SemaphoreType.DMA((4,)),
            pltpu.SemaphoreType.DMA((4,)),
        ],
    )
    ys = pl.pallas_call(
        _ffn_kernel,
        grid_spec=grid_spec,
        out_shape=jax.ShapeDtypeStruct((P, D), jnp.float32),
    )(be, brow, nact, uslot, dwait, ieid, islot, nxtg,
      xs, b1.reshape(E, 1, H), b2.reshape(E, 1, D), W1, W2)

    out2d = pl.kernel(
        _combine_body,
        out_type=jax.ShapeDtypeStruct((S, D), jnp.float32),
        mesh=_sc_mesh(),
        scratch_types=[
            pltpu.VMEM((CHUNK,), jnp.int32),
            pltpu.VMEM((CHUNK,), jnp.int32),
            pltpu.VMEM((CHUNK, D), jnp.float32),
            pltpu.VMEM((CHUNK, D), jnp.float32),
            pltpu.VMEM((CHUNK * EPAD,), jnp.float32),
            pltpu.SemaphoreType.DMA,
        ],
    )(ys, pos, probw.reshape(S * EPAD))

    return out2d.reshape(1, S, D)


def _full_unused():
    pass


# speedup vs baseline: 1.0062x; 1.0062x over previous
import functools

import jax
import jax.numpy as jnp
from jax.experimental import pallas as pl
from jax.experimental.pallas import tpu as pltpu
from jax.experimental.pallas import tpu_sc as plsc

D = 768
H = 3072
E = 8
S = 2048
EPAD = 128
NEG = -1e30

BLK = 512
G = 16
P = G * BLK

NC, NS = 2, 16
NW = NC * NS
CHUNK = S // NW

@functools.cache
def _sc_mesh():
    return plsc.VectorSubcoreMesh(core_axis_name="c", subcore_axis_name="s",
                                  num_cores=NC, num_subcores=NS)


def _route_kernel(x_ref, gw_ref, gb_ref, posw_ref, probw_ref, meta_ref):
    x = x_ref[...]
    scores = jnp.dot(x, gw_ref[...], preferred_element_type=jnp.float32) + gb_ref[...]
    lane = jax.lax.broadcasted_iota(jnp.int32, (S, EPAD), 1)
    m1 = jnp.max(scores, axis=1, keepdims=True)
    i1 = jnp.min(jnp.where(scores == m1, lane, EPAD), axis=1, keepdims=True)
    masked = jnp.where(lane == i1, NEG, scores)
    m2 = jnp.max(masked, axis=1, keepdims=True)
    i2 = jnp.min(jnp.where(masked == m2, lane, EPAD), axis=1, keepdims=True)
    e2 = jnp.exp(m2 - m1)
    p1 = 1.0 / (1.0 + e2)
    p2 = e2 / (1.0 + e2)

    oh = (jnp.where(lane == i1, 1.0, 0.0)
          + jnp.where(lane == i2 + E, 1.0, 0.0))
    r_i = jax.lax.broadcasted_iota(jnp.int32, (S, S), 0)
    c_i = jax.lax.broadcasted_iota(jnp.int32, (S, S), 1)
    tril = jnp.where(r_i >= c_i, 1.0, 0.0)
    csum = jnp.dot(tril, oh, preferred_element_type=jnp.float32)
    lane1 = jax.lax.broadcasted_iota(jnp.int32, (1, EPAD), 1)
    tot = csum[S - 1:S, :]
    t0 = jnp.where(lane1 < E, tot, 0.0)
    t1s = jnp.pad(tot[:, E:], ((0, 0), (0, E)))
    cnt = t0 + jnp.where(lane1 < E, t1s, 0.0)

    cnt_i = cnt.astype(jnp.int32)
    pc_i = ((cnt_i + (BLK - 1)) >> 9) << 9
    pc = pc_i.astype(jnp.float32)
    ru = jax.lax.broadcasted_iota(jnp.int32, (EPAD, EPAD), 0)
    cu = jax.lax.broadcasted_iota(jnp.int32, (EPAD, EPAD), 1)
    triu = jnp.where(ru < cu, 1.0, 0.0)
    off = jnp.dot(pc, triu, preferred_element_type=jnp.float32)
    cum = off + pc

    rank0 = jnp.sum(jnp.where(lane == i1, csum, 0.0), axis=1, keepdims=True) - 1.0
    rank1 = jnp.sum(jnp.where(lane == i2 + E, csum, 0.0), axis=1, keepdims=True) - 1.0
    off_b = jnp.broadcast_to(off, (S, EPAD))
    t0_b = jnp.broadcast_to(t0, (S, EPAD))
    off0 = jnp.sum(jnp.where(lane == i1, off_b, 0.0), axis=1, keepdims=True)
    off1 = jnp.sum(jnp.where(lane == i2, off_b, 0.0), axis=1, keepdims=True)
    t0e1 = jnp.sum(jnp.where(lane == i2, t0_b, 0.0), axis=1, keepdims=True)
    pos0 = off0 + rank0
    pos1 = off1 + t0e1 + rank1
    posw_ref[...] = jnp.where(
        lane == 0, pos0, jnp.where(lane == 1, pos1, 0.0)).astype(jnp.int32)
    probw_ref[...] = jnp.where(
        lane < 16, p1, jnp.where(lane < 32, p2, 0.0))

    total_i = jnp.sum(pc_i, axis=1, keepdims=True)
    gclamp = jnp.minimum(lane1 * BLK, total_i - 1)
    cum_i = cum.astype(jnp.int32)
    be = jnp.zeros((1, EPAD), jnp.int32)
    for e in range(E):
        be = be + jnp.where(cum_i[:, e:e + 1] <= gclamp, 1, 0)
    be = jnp.minimum(be, E - 1)
    nact = total_i >> 9
    brow = jnp.minimum(lane1, nact - 1)
    row = jax.lax.broadcasted_iota(jnp.int32, (8, EPAD), 0)
    meta_ref[...] = jnp.where(
        row == 0, jnp.broadcast_to(be, (8, EPAD)),
        jnp.where(row == 1, jnp.broadcast_to(brow, (8, EPAD)),
                  jnp.broadcast_to(nact, (8, EPAD))))


def _scatter_body(x_hbm, pos_hbm, xs_hbm, xv, i0v, i1v, sem):
    wid = jax.lax.axis_index("s") * NC + jax.lax.axis_index("c")
    base = wid * CHUNK
    pltpu.sync_copy(pos_hbm.at[0, pl.ds(base, CHUNK)], i0v)
    pltpu.sync_copy(pos_hbm.at[1, pl.ds(base, CHUNK)], i1v)
    pltpu.sync_copy(x_hbm.at[pl.ds(base, CHUNK)], xv)
    c0 = pltpu.async_copy(xv, xs_hbm.at[i0v], sem)
    c1 = pltpu.async_copy(xv, xs_hbm.at[i1v], sem)
    c0.wait()
    c1.wait()


def _ffn_kernel(be_ref, brow_ref, nact_ref, xs_ref, w1_ref, b1_ref,
                w2_ref, b2_ref, y_ref):
    g = pl.program_id(0)

    @pl.when(g < nact_ref[0])
    def _():
        hid = jnp.dot(xs_ref[...].astype(jnp.bfloat16),
                      w1_ref[0].astype(jnp.bfloat16),
                      preferred_element_type=jnp.float32) + b1_ref[0, 0]
        hid = jnp.maximum(hid, 0.0)
        y_ref[...] = jnp.dot(hid.astype(jnp.bfloat16),
                             w2_ref[0].astype(jnp.bfloat16),
                             preferred_element_type=jnp.float32) + b2_ref[0, 0]


def _combine_body(ys_hbm, pos_hbm, probw_hbm, out_hbm, i0v, i1v, y0v, y1v,
                  qwv, sem):
    wid = jax.lax.axis_index("s") * NC + jax.lax.axis_index("c")
    base = wid * CHUNK
    pltpu.sync_copy(pos_hbm.at[0, pl.ds(base, CHUNK)], i0v)
    pltpu.sync_copy(pos_hbm.at[1, pl.ds(base, CHUNK)], i1v)
    pltpu.sync_copy(probw_hbm.at[pl.ds(base * EPAD, CHUNK * EPAD)], qwv)
    cp0 = pltpu.async_copy(ys_hbm.at[i0v], y0v, sem)
    cp1 = pltpu.async_copy(ys_hbm.at[i1v], y1v, sem)
    cp0.wait()
    cp1.wait()

    def row_add(r, carry):
        p0 = qwv[pl.ds(r * EPAD, 16)]
        p1 = qwv[pl.ds(r * EPAD + 16, 16)]
        for j in range(D // 16):
            sl = pl.ds(j * 16, 16)
            y0v[r, sl] = y0v[r, sl] * p0 + y1v[r, sl] * p1
        return carry

    jax.lax.fori_loop(0, CHUNK, row_add, 0)
    pltpu.sync_copy(y0v, out_hbm.at[pl.ds(base, CHUNK)])


def kernel(x, gate_W, gate_b, W1, b1, W2, b2):
    x2d = x.reshape(S, D)
    gate_Wp = jnp.pad(gate_W, ((0, 0), (0, EPAD - E)))
    gate_bp = jnp.pad(gate_b.reshape(1, E), ((0, 0), (0, EPAD - E)),
                      constant_values=NEG)

    posw, probw, meta = pl.pallas_call(
        _route_kernel,
        out_shape=(
            jax.ShapeDtypeStruct((S, EPAD), jnp.int32),
            jax.ShapeDtypeStruct((S, EPAD), jnp.float32),
            jax.ShapeDtypeStruct((8, EPAD), jnp.int32),
        ),
    )(x2d, gate_Wp, gate_bp)

    pos = jnp.stack([posw[:, 0], posw[:, 1]])
    be = meta[0, :G]
    brow = meta[1, :G]
    nact = meta[2, :1]

    xs = pl.kernel(
        _scatter_body,
        out_type=jax.ShapeDtypeStruct((P, D), jnp.float32),
        mesh=_sc_mesh(),
        scratch_types=[
            pltpu.VMEM((CHUNK, D), jnp.float32),
            pltpu.VMEM((CHUNK,), jnp.int32),
            pltpu.VMEM((CHUNK,), jnp.int32),
            pltpu.SemaphoreType.DMA,
        ],
    )(x2d, pos)

    grid_spec = pltpu.PrefetchScalarGridSpec(
        num_scalar_prefetch=3,
        grid=(G,),
        in_specs=[
            pl.BlockSpec((BLK, D), lambda g, be, br, na: (br[g], 0)),
            pl.BlockSpec((1, D, H), lambda g, be, br, na: (be[g], 0, 0)),
            pl.BlockSpec((1, 1, H), lambda g, be, br, na: (be[g], 0, 0)),
            pl.BlockSpec((1, H, D), lambda g, be, br, na: (be[g], 0, 0)),
            pl.BlockSpec((1, 1, D), lambda g, be, br, na: (be[g], 0, 0)),
        ],
        out_specs=pl.BlockSpec((BLK, D), lambda g, be, br, na: (br[g], 0)),
    )
    ys = pl.pallas_call(
        _ffn_kernel,
        grid_spec=grid_spec,
        out_shape=jax.ShapeDtypeStruct((P, D), jnp.float32),
    )(be, brow, nact, xs, W1, b1.reshape(E, 1, H), W2,
      b2.reshape(E, 1, D))

    out2d = pl.kernel(
        _combine_body,
        out_type=jax.ShapeDtypeStruct((S, D), jnp.float32),
        mesh=_sc_mesh(),
        scratch_types=[
            pltpu.VMEM((CHUNK,), jnp.int32),
            pltpu.VMEM((CHUNK,), jnp.int32),
            pltpu.VMEM((CHUNK, D), jnp.float32),
            pltpu.VMEM((CHUNK, D), jnp.float32),
            pltpu.VMEM((CHUNK * EPAD,), jnp.float32),
            pltpu.SemaphoreType.DMA,
        ],
    )(ys, pos, probw.reshape(S * EPAD))

    return out2d.reshape(1, S, D)


def _full_unused():
    pass
